# Initial kernel scaffold; baseline (speedup 1.0000x reference)
import jax
import jax.numpy as jnp
from jax.experimental import pallas as pl


def kernel(map_t, pos, action_mask, h0, c0, seq_lens, edge_index, W1, Ws1, b1, W2, Ws2, b2, Wi, Wh, bi, bh, P1, pb1, P2, pb2, P3, pb3):
    raise NotImplementedError("write your pallas kernel here")


# speedup vs baseline: 35.4673x; 35.4673x over previous
import functools

import jax
import jax.numpy as jnp
from jax import lax
from jax.experimental import pallas as pl
from jax.experimental.pallas import tpu as pltpu
from jax.experimental.pallas import tpu_sc as plsc

B = 512
S = 12
F = 128
H = 256
A = 64
NCELL = S * S
NW = 32
BPW = B // NW
K = 13
KA = 8
KB = K - KA

OFFS = [(0, 0), (-1, 0), (1, 0), (0, -1), (0, 1), (-2, 0), (2, 0),
        (0, -2), (0, 2), (-1, -1), (-1, 1), (1, -1), (1, 1)]
PART = {1: (5, 9, 10), 2: (6, 11, 12), 3: (7, 9, 11), 4: (8, 10, 12)}

BBLK = 64


def _sc_gather(xflat, rpos, cpos):
    mesh = plsc.VectorSubcoreMesh(core_axis_name="c", subcore_axis_name="s")

    @functools.partial(
        pl.kernel,
        mesh=mesh,
        out_type=jax.ShapeDtypeStruct((K * B, F), jnp.float32),
        scratch_types=[
            pltpu.VMEM((KA * BPW,), jnp.int32),
            pltpu.VMEM((KB * BPW,), jnp.int32),
            pltpu.VMEM((KA * BPW, F), jnp.float32),
            pltpu.VMEM((KB * BPW, F), jnp.float32),
            pltpu.VMEM((BPW,), jnp.int32),
            pltpu.VMEM((BPW,), jnp.int32),
            pltpu.SemaphoreType.DMA,
        ],
    )
    def gather_k(x_hbm, r_hbm, c_hbm, out_hbm,
                 idx_a, idx_b, rows_a, rows_b, r_v, c_v, sem):
        nc = 2
        wid = lax.axis_index("s") * nc + lax.axis_index("c")
        base = wid * BPW
        pltpu.sync_copy(r_hbm.at[pl.ds(base, BPW)], r_v)
        pltpu.sync_copy(c_hbm.at[pl.ds(base, BPW)], c_v)
        rv = r_v[...]
        cv = c_v[...]
        b_ids = (base + lax.iota(jnp.int32, BPW)) * NCELL
        for k, (dr, dc) in enumerate(OFFS):
            rr = jnp.clip(rv + dr, 0, S - 1)
            cc = jnp.clip(cv + dc, 0, S - 1)
            idx16 = b_ids + rr * S + cc
            if k < KA:
                idx_a[pl.ds(k * BPW, BPW)] = idx16
            else:
                idx_b[pl.ds((k - KA) * BPW, BPW)] = idx16
        d1 = pltpu.async_copy(x_hbm.at[idx_a], rows_a, sem)
        d2 = pltpu.async_copy(x_hbm.at[idx_b], rows_b, sem)
        d1.wait()
        d2.wait()
        for k in range(K):
            src = rows_a if k < KA else rows_b
            off = (k if k < KA else k - KA) * BPW
            pltpu.sync_copy(src.at[pl.ds(off, BPW)],
                            out_hbm.at[pl.ds(k * B + base, BPW)])

    return gather_k(xflat, rpos, cpos)


def _tc_body(map_ref, picked_ref, pos_ref, am_ref, h0_ref, c0_ref,
             w1_ref, ws1_ref, b1_ref, w2_ref, ws2_ref, b2_ref,
             wi_ref, wh_ref, bi_ref, bh_ref,
             p1_ref, pb1_ref, p2_ref, pb2_ref, p3_ref, pb3_ref,
             out_ref):
    f32 = jnp.float32
    x3 = map_ref[...]
    meta_sum = jnp.sum(x3, axis=1)

    r = pos_ref[:, 0:1]
    c = pos_ref[:, 1:2]
    g = []
    masks = []
    for k, (dr, dc) in enumerate(OFFS):
        valid = ((r + dr >= 0) & (r + dr < S) & (c + dc >= 0) & (c + dc < S))
        mk = valid.astype(f32)
        masks.append(mk)
        g.append(picked_ref[k] * mk)

    m_c = g[1] + g[2] + g[3] + g[4]
    ms = [m_c] + [g[0] + g[a] + g[b_] + g[c_] for (a, b_, c_) in PART.values()]
    xs = [g[0], g[1], g[2], g[3], g[4]]
    m5 = jnp.concatenate(ms, axis=0)
    x5 = jnp.concatenate(xs, axis=0)

    w1 = w1_ref[...]
    b1 = b1_ref[...]
    h5 = jnp.maximum(
        jnp.dot(m5, w1, preferred_element_type=f32)
        + jnp.dot(x5, ws1_ref[...], preferred_element_type=f32) + b1, 0.0)
    h_meta = jnp.maximum(
        jnp.dot(meta_sum, w1, preferred_element_type=f32) + b1, 0.0)

    hj = h5[0:BBLK]
    m2 = h_meta
    for i in range(4):
        m2 = m2 + h5[(1 + i) * BBLK:(2 + i) * BBLK] * masks[1 + i]
    h2 = jnp.maximum(
        jnp.dot(m2, w2_ref[...], preferred_element_type=f32)
        + jnp.dot(hj, ws2_ref[...], preferred_element_type=f32)
        + b2_ref[...], 0.0)

    gates = (jnp.dot(h2, wi_ref[...], preferred_element_type=f32)
             + jnp.dot(h0_ref[...], wh_ref[...], preferred_element_type=f32)
             + bi_ref[...] + bh_ref[...])
    i_g = gates[:, 0 * H:1 * H]
    f_g = gates[:, 1 * H:2 * H]
    g_g = gates[:, 2 * H:3 * H]
    o_g = gates[:, 3 * H:4 * H]
    cst = (jax.nn.sigmoid(f_g) * c0_ref[...]
           + jax.nn.sigmoid(i_g) * jnp.tanh(g_g))
    hl = jax.nn.sigmoid(o_g) * jnp.tanh(cst)

    def elu(v):
        return jnp.where(v > 0, v, jnp.exp(jnp.minimum(v, 0.0)) - 1.0)

    l = elu(jnp.dot(hl, p1_ref[...], preferred_element_type=f32) + pb1_ref[...])
    l = elu(jnp.dot(l, p2_ref[...], preferred_element_type=f32) + pb2_ref[...])
    logits = jnp.dot(l, p3_ref[...], preferred_element_type=f32) + pb3_ref[...]

    am = am_ref[...].astype(f32)
    inf_mask = jnp.maximum(jnp.log(am), jnp.finfo(f32).min)
    out_ref[...] = logits + inf_mask


def _tc_call(map_x, picked, pos, action_mask, h0, c0, weights):
    def bcast(shape):
        nd = len(shape)
        return pl.BlockSpec(shape, lambda i, _n=nd: (0,) * _n)

    in_specs = [
        pl.BlockSpec((BBLK, NCELL, F), lambda i: (i, 0, 0)),
        pl.BlockSpec((K, BBLK, F), lambda i: (0, i, 0)),
        pl.BlockSpec((BBLK, 2), lambda i: (i, 0)),
        pl.BlockSpec((BBLK, A), lambda i: (i, 0)),
        pl.BlockSpec((BBLK, H), lambda i: (i, 0)),
        pl.BlockSpec((BBLK, H), lambda i: (i, 0)),
    ] + [bcast(w.shape) for w in weights]
    return pl.pallas_call(
        _tc_body,
        grid=(B // BBLK,),
        in_specs=in_specs,
        out_specs=pl.BlockSpec((BBLK, A), lambda i: (i, 0)),
        out_shape=jax.ShapeDtypeStruct((B, A), jnp.float32),
    )(map_x, picked, pos, action_mask, h0, c0, *weights)


def kernel(map_t, pos, action_mask, h0, c0, seq_lens, edge_index,
           W1, Ws1, b1, W2, Ws2, b2, Wi, Wh, bi, bh,
           P1, pb1, P2, pb2, P3, pb3):
    del seq_lens, edge_index
    xflat = map_t.reshape(B * NCELL, F)
    pos = pos.astype(jnp.int32)
    rpos = pos[:, 0]
    cpos = pos[:, 1]

    picked = _sc_gather(xflat, rpos, cpos).reshape(K, B, F)

    map_x = map_t.reshape(B, NCELL, F)
    weights = (W1, Ws1, b1.reshape(1, F), W2, Ws2, b2.reshape(1, F),
               Wi, Wh, bi.reshape(1, 4 * H), bh.reshape(1, 4 * H),
               P1, pb1.reshape(1, 256), P2, pb2.reshape(1, 256),
               P3, pb3.reshape(1, A))
    return _tc_call(map_x, picked, pos, action_mask.astype(jnp.int32),
                    h0, c0, weights)


# speedup vs baseline: 52.8255x; 1.4894x over previous
import functools

import jax
import jax.numpy as jnp
from jax import lax
from jax.experimental import pallas as pl
from jax.experimental.pallas import tpu as pltpu
from jax.experimental.pallas import tpu_sc as plsc

B = 512
S = 12
F = 128
H = 256
A = 64
NCELL = S * S
NW = 32
BPW = B // NW
K = 13
KA = 8
KB = K - KA

OFFS = [(0, 0), (-1, 0), (1, 0), (0, -1), (0, 1), (-2, 0), (2, 0),
        (0, -2), (0, 2), (-1, -1), (-1, 1), (1, -1), (1, 1)]
PART = {1: (5, 9, 10), 2: (6, 11, 12), 3: (7, 9, 11), 4: (8, 10, 12)}

BBLKA = 64
BBLKB = 128


def _tca_body(map_ref, xout_ref, meta_ref):
    x4 = map_ref[...]
    meta_ref[...] = jnp.sum(x4, axis=(1, 2))
    for r in range(S):
        xout_ref[:, r * S:(r + 1) * S, :] = x4[:, r]


def _tca_call(map_t):
    return pl.pallas_call(
        _tca_body,
        grid=(B // BBLKA,),
        in_specs=[pl.BlockSpec((BBLKA, S, S, F), lambda i: (i, 0, 0, 0))],
        out_specs=[
            pl.BlockSpec((BBLKA, NCELL, F), lambda i: (i, 0, 0)),
            pl.BlockSpec((BBLKA, F), lambda i: (i, 0)),
        ],
        out_shape=[
            jax.ShapeDtypeStruct((B, NCELL, F), jnp.float32),
            jax.ShapeDtypeStruct((B, F), jnp.float32),
        ],
    )(map_t)


def _sc_gather(xflat, rpos, cpos):
    mesh = plsc.VectorSubcoreMesh(core_axis_name="c", subcore_axis_name="s")

    @functools.partial(
        pl.kernel,
        mesh=mesh,
        out_type=jax.ShapeDtypeStruct((K * B, F), jnp.float32),
        scratch_types=[
            pltpu.VMEM((KA * BPW,), jnp.int32),
            pltpu.VMEM((KB * BPW,), jnp.int32),
            pltpu.VMEM((KA * BPW, F), jnp.float32),
            pltpu.VMEM((KB * BPW, F), jnp.float32),
            pltpu.VMEM((BPW,), jnp.int32),
            pltpu.VMEM((BPW,), jnp.int32),
            pltpu.SemaphoreType.DMA,
        ],
    )
    def gather_k(x_hbm, r_hbm, c_hbm, out_hbm,
                 idx_a, idx_b, rows_a, rows_b, r_v, c_v, sem):
        nc = 2
        wid = lax.axis_index("s") * nc + lax.axis_index("c")
        base = wid * BPW
        pltpu.sync_copy(r_hbm.at[pl.ds(base, BPW)], r_v)
        pltpu.sync_copy(c_hbm.at[pl.ds(base, BPW)], c_v)
        rv = r_v[...]
        cv = c_v[...]
        b_ids = (base + lax.iota(jnp.int32, BPW)) * NCELL
        for k, (dr, dc) in enumerate(OFFS):
            rr = jnp.clip(rv + dr, 0, S - 1)
            cc = jnp.clip(cv + dc, 0, S - 1)
            idx16 = b_ids + rr * S + cc
            if k < KA:
                idx_a[pl.ds(k * BPW, BPW)] = idx16
            else:
                idx_b[pl.ds((k - KA) * BPW, BPW)] = idx16
        d1 = pltpu.async_copy(x_hbm.at[idx_a], rows_a, sem)
        d2 = pltpu.async_copy(x_hbm.at[idx_b], rows_b, sem)
        d1.wait()
        d2.wait()
        for k in range(K):
            src = rows_a if k < KA else rows_b
            off = (k if k < KA else k - KA) * BPW
            pltpu.sync_copy(src.at[pl.ds(off, BPW)],
                            out_hbm.at[pl.ds(k * B + base, BPW)])

    return gather_k(xflat, rpos, cpos)


def _tcb_body(picked_ref, meta_ref, pos_ref, am_ref, h0_ref, c0_ref,
              w1_ref, ws1_ref, b1_ref, w2_ref, ws2_ref, b2_ref,
              wi_ref, wh_ref, bi_ref, bh_ref,
              p1_ref, pb1_ref, p2_ref, pb2_ref, p3_ref, pb3_ref,
              out_ref):
    f32 = jnp.float32
    nb = BBLKB
    meta_sum = meta_ref[...]

    r = pos_ref[:, 0:1]
    c = pos_ref[:, 1:2]
    g = []
    masks = []
    for k, (dr, dc) in enumerate(OFFS):
        valid = ((r + dr >= 0) & (r + dr < S) & (c + dc >= 0) & (c + dc < S))
        mk = valid.astype(f32)
        masks.append(mk)
        g.append(picked_ref[k] * mk)

    m_c = g[1] + g[2] + g[3] + g[4]
    ms = [m_c] + [g[0] + g[a] + g[b_] + g[c_] for (a, b_, c_) in PART.values()]
    xs = [g[0], g[1], g[2], g[3], g[4]]
    m5 = jnp.concatenate(ms, axis=0)
    x5 = jnp.concatenate(xs, axis=0)

    w1 = w1_ref[...]
    b1 = b1_ref[...]
    h5 = jnp.maximum(
        jnp.dot(m5, w1, preferred_element_type=f32)
        + jnp.dot(x5, ws1_ref[...], preferred_element_type=f32) + b1, 0.0)
    h_meta = jnp.maximum(
        jnp.dot(meta_sum, w1, preferred_element_type=f32) + b1, 0.0)

    hj = h5[0:nb]
    m2 = h_meta
    for i in range(4):
        m2 = m2 + h5[(1 + i) * nb:(2 + i) * nb] * masks[1 + i]
    h2 = jnp.maximum(
        jnp.dot(m2, w2_ref[...], preferred_element_type=f32)
        + jnp.dot(hj, ws2_ref[...], preferred_element_type=f32)
        + b2_ref[...], 0.0)

    gates = (jnp.dot(h2, wi_ref[...], preferred_element_type=f32)
             + jnp.dot(h0_ref[...], wh_ref[...], preferred_element_type=f32)
             + bi_ref[...] + bh_ref[...])
    i_g = gates[:, 0 * H:1 * H]
    f_g = gates[:, 1 * H:2 * H]
    g_g = gates[:, 2 * H:3 * H]
    o_g = gates[:, 3 * H:4 * H]
    cst = (jax.nn.sigmoid(f_g) * c0_ref[...]
           + jax.nn.sigmoid(i_g) * jnp.tanh(g_g))
    hl = jax.nn.sigmoid(o_g) * jnp.tanh(cst)

    def elu(v):
        return jnp.where(v > 0, v, jnp.exp(jnp.minimum(v, 0.0)) - 1.0)

    l = elu(jnp.dot(hl, p1_ref[...], preferred_element_type=f32) + pb1_ref[...])
    l = elu(jnp.dot(l, p2_ref[...], preferred_element_type=f32) + pb2_ref[...])
    logits = jnp.dot(l, p3_ref[...], preferred_element_type=f32) + pb3_ref[...]

    am = am_ref[...].astype(f32)
    inf_mask = jnp.maximum(jnp.log(am), jnp.finfo(f32).min)
    out_ref[...] = logits + inf_mask


def _tcb_call(picked, meta, pos, action_mask, h0, c0, weights):
    def bcast(shape):
        nd = len(shape)
        return pl.BlockSpec(shape, lambda i, _n=nd: (0,) * _n)

    in_specs = [
        pl.BlockSpec((K, BBLKB, F), lambda i: (0, i, 0)),
        pl.BlockSpec((BBLKB, F), lambda i: (i, 0)),
        pl.BlockSpec((BBLKB, 2), lambda i: (i, 0)),
        pl.BlockSpec((BBLKB, A), lambda i: (i, 0)),
        pl.BlockSpec((BBLKB, H), lambda i: (i, 0)),
        pl.BlockSpec((BBLKB, H), lambda i: (i, 0)),
    ] + [bcast(w.shape) for w in weights]
    return pl.pallas_call(
        _tcb_body,
        grid=(B // BBLKB,),
        in_specs=in_specs,
        out_specs=pl.BlockSpec((BBLKB, A), lambda i: (i, 0)),
        out_shape=jax.ShapeDtypeStruct((B, A), jnp.float32),
    )(picked, meta, pos, action_mask, h0, c0, *weights)


def kernel(map_t, pos, action_mask, h0, c0, seq_lens, edge_index,
           W1, Ws1, b1, W2, Ws2, b2, Wi, Wh, bi, bh,
           P1, pb1, P2, pb2, P3, pb3):
    del seq_lens, edge_index
    pos = pos.astype(jnp.int32)
    rpos = pos[:, 0]
    cpos = pos[:, 1]

    xcompact, meta = _tca_call(map_t)
    picked = _sc_gather(xcompact.reshape(B * NCELL, F), rpos, cpos)
    picked = picked.reshape(K, B, F)

    weights = (W1, Ws1, b1.reshape(1, F), W2, Ws2, b2.reshape(1, F),
               Wi, Wh, bi.reshape(1, 4 * H), bh.reshape(1, 4 * H),
               P1, pb1.reshape(1, 256), P2, pb2.reshape(1, 256),
               P3, pb3.reshape(1, A))
    return _tcb_call(picked, meta, pos, action_mask.astype(jnp.int32),
                     h0, c0, weights)


# speedup vs baseline: 126.7712x; 2.3998x over previous
import functools

import jax
import jax.numpy as jnp
from jax import lax
from jax.experimental import pallas as pl
from jax.experimental.pallas import tpu as pltpu
from jax.experimental.pallas import tpu_sc as plsc

B = 512
S = 12
F = 128
H = 256
A = 64
NCELL = S * S
NW = 32
BPW = B // NW
K = 13
KA = 8
KB = K - KA

OFFS = [(0, 0), (-1, 0), (1, 0), (0, -1), (0, 1), (-2, 0), (2, 0),
        (0, -2), (0, 2), (-1, -1), (-1, 1), (1, -1), (1, 1)]
PART = {1: (5, 9, 10), 2: (6, 11, 12), 3: (7, 9, 11), 4: (8, 10, 12)}

BBLK = 128


def _sc_gather(xflat, rpos, cpos):
    mesh = plsc.VectorSubcoreMesh(core_axis_name="c", subcore_axis_name="s")

    @functools.partial(
        pl.kernel,
        mesh=mesh,
        out_type=jax.ShapeDtypeStruct((K * B, F), jnp.float32),
        scratch_types=[
            pltpu.VMEM((KA * BPW,), jnp.int32),
            pltpu.VMEM((KB * BPW,), jnp.int32),
            pltpu.VMEM((KA * BPW, F), jnp.float32),
            pltpu.VMEM((KB * BPW, F), jnp.float32),
            pltpu.VMEM((BPW,), jnp.int32),
            pltpu.VMEM((BPW,), jnp.int32),
            pltpu.SemaphoreType.DMA,
        ],
    )
    def gather_k(x_hbm, r_hbm, c_hbm, out_hbm,
                 idx_a, idx_b, rows_a, rows_b, r_v, c_v, sem):
        nc = 2
        wid = lax.axis_index("s") * nc + lax.axis_index("c")
        base = wid * BPW
        pltpu.sync_copy(r_hbm.at[pl.ds(base, BPW)], r_v)
        pltpu.sync_copy(c_hbm.at[pl.ds(base, BPW)], c_v)
        rv = r_v[...]
        cv = c_v[...]
        b_ids = base + lax.iota(jnp.int32, BPW)
        for k, (dr, dc) in enumerate(OFFS):
            rr = jnp.clip(rv + dr, 0, S - 1)
            cc = jnp.clip(cv + dc, 0, S - 1)
            idx16 = (rr * S + cc) * B + b_ids
            if k < KA:
                idx_a[pl.ds(k * BPW, BPW)] = idx16
            else:
                idx_b[pl.ds((k - KA) * BPW, BPW)] = idx16
        d1 = pltpu.async_copy(x_hbm.at[idx_a], rows_a, sem)
        d2 = pltpu.async_copy(x_hbm.at[idx_b], rows_b, sem)
        d1.wait()
        d2.wait()
        for k in range(K):
            src = rows_a if k < KA else rows_b
            off = (k if k < KA else k - KA) * BPW
            pltpu.sync_copy(src.at[pl.ds(off, BPW)],
                            out_hbm.at[pl.ds(k * B + base, BPW)])

    return gather_k(xflat, rpos, cpos)


def _tc_body(map_ref, picked_ref, pos_ref, am_ref, h0_ref, c0_ref,
             w1_ref, ws1_ref, b1_ref, w2_ref, ws2_ref, b2_ref,
             wi_ref, wh_ref, bi_ref, bh_ref,
             p1_ref, pb1_ref, p2_ref, pb2_ref, p3_ref, pb3_ref,
             out_ref):
    f32 = jnp.float32
    nb = BBLK
    x4 = map_ref[...]
    meta_sum = jnp.sum(x4, axis=(0, 1))

    r = pos_ref[:, 0:1]
    c = pos_ref[:, 1:2]
    g = []
    masks = []
    for k, (dr, dc) in enumerate(OFFS):
        valid = ((r + dr >= 0) & (r + dr < S) & (c + dc >= 0) & (c + dc < S))
        mk = valid.astype(f32)
        masks.append(mk)
        g.append(picked_ref[k] * mk)

    m_c = g[1] + g[2] + g[3] + g[4]
    ms = [m_c] + [g[0] + g[a] + g[b_] + g[c_] for (a, b_, c_) in PART.values()]
    xs = [g[0], g[1], g[2], g[3], g[4]]
    m5 = jnp.concatenate(ms, axis=0)
    x5 = jnp.concatenate(xs, axis=0)

    w1 = w1_ref[...]
    b1 = b1_ref[...]
    h5 = jnp.maximum(
        jnp.dot(m5, w1, preferred_element_type=f32)
        + jnp.dot(x5, ws1_ref[...], preferred_element_type=f32) + b1, 0.0)
    h_meta = jnp.maximum(
        jnp.dot(meta_sum, w1, preferred_element_type=f32) + b1, 0.0)

    hj = h5[0:nb]
    m2 = h_meta
    for i in range(4):
        m2 = m2 + h5[(1 + i) * nb:(2 + i) * nb] * masks[1 + i]
    h2 = jnp.maximum(
        jnp.dot(m2, w2_ref[...], preferred_element_type=f32)
        + jnp.dot(hj, ws2_ref[...], preferred_element_type=f32)
        + b2_ref[...], 0.0)

    gates = (jnp.dot(h2, wi_ref[...], preferred_element_type=f32)
             + jnp.dot(h0_ref[...], wh_ref[...], preferred_element_type=f32)
             + bi_ref[...] + bh_ref[...])
    i_g = gates[:, 0 * H:1 * H]
    f_g = gates[:, 1 * H:2 * H]
    g_g = gates[:, 2 * H:3 * H]
    o_g = gates[:, 3 * H:4 * H]
    cst = (jax.nn.sigmoid(f_g) * c0_ref[...]
           + jax.nn.sigmoid(i_g) * jnp.tanh(g_g))
    hl = jax.nn.sigmoid(o_g) * jnp.tanh(cst)

    def elu(v):
        return jnp.where(v > 0, v, jnp.exp(jnp.minimum(v, 0.0)) - 1.0)

    l = elu(jnp.dot(hl, p1_ref[...], preferred_element_type=f32) + pb1_ref[...])
    l = elu(jnp.dot(l, p2_ref[...], preferred_element_type=f32) + pb2_ref[...])
    logits = jnp.dot(l, p3_ref[...], preferred_element_type=f32) + pb3_ref[...]

    am = am_ref[...].astype(f32)
    inf_mask = jnp.maximum(jnp.log(am), jnp.finfo(f32).min)
    out_ref[...] = logits + inf_mask


def _tc_call(map_cm, picked, pos, action_mask, h0, c0, weights):
    def bcast(shape):
        nd = len(shape)
        return pl.BlockSpec(shape, lambda i, _n=nd: (0,) * _n)

    in_specs = [
        pl.BlockSpec((S, S, BBLK, F), lambda i: (0, 0, i, 0)),
        pl.BlockSpec((K, BBLK, F), lambda i: (0, i, 0)),
        pl.BlockSpec((BBLK, 2), lambda i: (i, 0)),
        pl.BlockSpec((BBLK, A), lambda i: (i, 0)),
        pl.BlockSpec((BBLK, H), lambda i: (i, 0)),
        pl.BlockSpec((BBLK, H), lambda i: (i, 0)),
    ] + [bcast(w.shape) for w in weights]
    return pl.pallas_call(
        _tc_body,
        grid=(B // BBLK,),
        in_specs=in_specs,
        out_specs=pl.BlockSpec((BBLK, A), lambda i: (i, 0)),
        out_shape=jax.ShapeDtypeStruct((B, A), jnp.float32),
    )(map_cm, picked, pos, action_mask, h0, c0, *weights)


def kernel(map_t, pos, action_mask, h0, c0, seq_lens, edge_index,
           W1, Ws1, b1, W2, Ws2, b2, Wi, Wh, bi, bh,
           P1, pb1, P2, pb2, P3, pb3):
    del seq_lens, edge_index
    pos = pos.astype(jnp.int32)
    rpos = pos[:, 0]
    cpos = pos[:, 1]

    map_cm = jnp.transpose(map_t, (1, 2, 0, 3))
    picked = _sc_gather(map_cm.reshape(NCELL * B, F), rpos, cpos)
    picked = picked.reshape(K, B, F)

    weights = (W1, Ws1, b1.reshape(1, F), W2, Ws2, b2.reshape(1, F),
               Wi, Wh, bi.reshape(1, 4 * H), bh.reshape(1, 4 * H),
               P1, pb1.reshape(1, 256), P2, pb2.reshape(1, 256),
               P3, pb3.reshape(1, A))
    return _tc_call(map_cm, picked, pos, action_mask.astype(jnp.int32),
                    h0, c0, weights)
